# baseline (speedup 1.0000x reference)
import functools

import jax
import jax.numpy as jnp
from jax import lax
from jax.experimental import pallas as pl
from jax.experimental.pallas import tpu as pltpu
from jax.experimental.pallas import tpu_sc as plsc

_B = 10000
_DEG = 32
_D = 128
_NC = 2
_NS = 16
_NW = _NC * _NS
_RPW = 320
_B_PAD = _NW * _RPW
_NBUF = 4
_NG = _RPW // _NBUF


def _weights_body(xt_ref, idxm_ref, sc_ref):
    x = xt_ref[...]
    rows = lax.broadcasted_iota(jnp.int32, x.shape, 0)
    dup = jnp.zeros(x.shape, jnp.bool_)
    for k in range(_DEG - 1):
        dup = jnp.logical_or(
            dup, jnp.logical_and(x == x[k : k + 1, :], rows > k)
        )
    first = jnp.logical_not(dup)
    u = jnp.sum(first.astype(jnp.float32), axis=0, keepdims=True)
    idxm_ref[...] = jnp.where(first, x, 0)
    inv = 1.0 / u
    sc_ref[...] = jnp.concatenate([inv, (_DEG - u) * inv], axis=0)


def _weights_tc(xt):
    return pl.pallas_call(
        _weights_body,
        out_shape=[
            jax.ShapeDtypeStruct((_DEG, _B_PAD), jnp.int32),
            jax.ShapeDtypeStruct((2, _B_PAD), jnp.float32),
        ],
    )(xt)


@functools.partial(
    pl.kernel,
    out_type=jax.ShapeDtypeStruct((_B_PAD, _D), jnp.float32),
    mesh=plsc.VectorSubcoreMesh(core_axis_name="c", subcore_axis_name="s"),
    scratch_types=[
        pltpu.VMEM((_RPW, _DEG), jnp.int32),
        pltpu.VMEM((_RPW, 16), jnp.float32),
        pltpu.VMEM((_D,), jnp.float32),
        pltpu.VMEM((_NBUF, _D), jnp.float32),
        pltpu.VMEM((_NBUF, _DEG, _D), jnp.float32),
        pltpu.SemaphoreType.DMA,
        pltpu.SemaphoreType.DMA,
        pltpu.SemaphoreType.DMA,
        pltpu.SemaphoreType.DMA,
        pltpu.SemaphoreType.DMA,
        pltpu.SemaphoreType.DMA,
        pltpu.SemaphoreType.DMA,
        pltpu.SemaphoreType.DMA,
    ],
)
def _sc_aggregate(idx_hbm, aux_hbm, feat_hbm, out_hbm,
                  idx_v, aux_v, fz_v, obuf, gbuf, *sems):
    gsems, osems = sems[:_NBUF], sems[_NBUF:]
    wid = lax.axis_index("s") * _NC + lax.axis_index("c")
    base = wid * _RPW
    pltpu.sync_copy(idx_hbm.at[pl.ds(base, _RPW)], idx_v)
    pltpu.sync_copy(aux_hbm.at[pl.ds(base, _RPW)], aux_v)
    pltpu.sync_copy(feat_hbm.at[0], fz_v)
    fz = [fz_v[pl.ds(d * 16, 16)] for d in range(_D // 16)]

    def _gather(row, b):
        return pltpu.make_async_copy(
            feat_hbm.at[idx_v.at[row]], gbuf.at[b], gsems[b]
        )

    def _put(row, b):
        return pltpu.make_async_copy(
            obuf.at[b], out_hbm.at[base + row], osems[b]
        )

    for b in range(_NBUF):
        _gather(b, b).start()

    def body(g, carry):
        for b in range(_NBUF):
            row = g * _NBUF + b
            _gather(row, b).wait()
            av = aux_v[row]
            scale, comp = av[0], av[1]

            @pl.when(g > 0)
            def _():
                _put(row - _NBUF, b).wait()

            for d in range(_D // 16):
                terms = [gbuf[b, j, pl.ds(d * 16, 16)] for j in range(_DEG)]
                while len(terms) > 1:
                    terms = [terms[i] + terms[i + 1]
                             for i in range(0, len(terms), 2)]
                obuf[b, pl.ds(d * 16, 16)] = terms[0] * scale - comp * fz[d]
            _put(row, b).start()
            nxt = row + _NBUF

            @pl.when(nxt < _RPW)
            def _():
                _gather(nxt, b).start()

        return carry

    lax.fori_loop(0, _NG, body, 0)
    for b in range(_NBUF):
        _put(_RPW - _NBUF + b, b).wait()


def kernel(nodes_real, to_neighs, features):
    del nodes_real
    idx_pad = jnp.pad(to_neighs, ((0, _B_PAD - _B), (0, 0)))
    idxm_t, sc = _weights_tc(idx_pad.T)
    aux = jnp.pad(sc.T, ((0, 0), (0, 14)))
    out = _sc_aggregate(idxm_t.T, aux, features)
    return out[:_B]
